# Initial kernel scaffold; baseline (speedup 1.0000x reference)
import jax
import jax.numpy as jnp
from jax.experimental import pallas as pl


def kernel(nodes, edges, semb, graph, bgraph, bweights, sgraph, sweights, snode_W1, snode_b1, snode_W2, snode_b2, sedge_W1, sedge_b1, sedge_W2, sedge_b2, cls_W1, cls_b1, cls_W2, cls_b2, cls_W3, cls_b3, bn_g, bn_b):
    raise NotImplementedError("write your pallas kernel here")


# speedup vs baseline: 1.4986x; 1.4986x over previous
import functools

import jax
import jax.numpy as jnp
from jax import lax
from jax.experimental import pallas as pl
from jax.experimental.pallas import tpu as pltpu
from jax.experimental.pallas import tpu_sc as plsc

NC, NS, LANES = 2, 16, 16
NW = NC * NS
BE = 40000
BEP = 40960
CHUNKS = 10
CW = 128
PER_W = CHUNKS * CW
D = 128
SP = 1024
ROWS_PER_SUB = SP // NS
TILE = 512
NT = BEP // TILE

_mesh = plsc.VectorSubcoreMesh(core_axis_name="c", subcore_axis_name="s")


def _gather_body(table_hbm, idx_hbm, out_hbm, idx_v, rows_v, gsem, wsem):
  wid = lax.axis_index("s") * NC + lax.axis_index("c")
  pltpu.sync_copy(idx_hbm.at[wid], idx_v)
  g = [None, None]
  w = [None, None]
  g[0] = pltpu.async_copy(table_hbm.at[idx_v.at[0]], rows_v.at[0], gsem)
  for j in range(CHUNKS):
    b, nb = j % 2, (j + 1) % 2
    g[b].wait()
    if j + 1 < CHUNKS:
      if w[nb] is not None:
        w[nb].wait()
      g[nb] = pltpu.async_copy(table_hbm.at[idx_v.at[j + 1]], rows_v.at[nb],
                               gsem)
    w[b] = pltpu.async_copy(
        rows_v.at[b], out_hbm.at[pl.ds(wid * PER_W + j * CW, CW)], wsem)
  for h in w:
    if h is not None:
      h.wait()


def _sc_gather(table, idx_3d):
  k = functools.partial(
      pl.kernel,
      mesh=_mesh,
      out_type=jax.ShapeDtypeStruct((BEP, D), jnp.float32),
      scratch_types=[
          pltpu.VMEM((CHUNKS, CW), jnp.int32),
          pltpu.VMEM((2, CW, D), jnp.float32),
          pltpu.SemaphoreType.DMA,
          pltpu.SemaphoreType.DMA,
      ],
  )(_gather_body)
  return k(table, idx_3d)


def _scatter_body(msgs_hbm, idx_hbm, zeros_hbm, out_hbm, idx_v, rows_v, acc):
  c = lax.axis_index("c")
  s = lax.axis_index("s")
  wid = s * NC + c
  pltpu.sync_copy(zeros_hbm, acc.at[pl.ds(s * ROWS_PER_SUB, ROWS_PER_SUB)])
  pltpu.sync_copy(idx_hbm.at[wid], idx_v)
  plsc.subcore_barrier()
  for j in range(CHUNKS):
    pltpu.sync_copy(msgs_hbm.at[pl.ds(wid * PER_W + j * CW, CW)], rows_v)
    pltpu.sync_copy(rows_v, acc.at[idx_v.at[j]], add=True)
  plsc.subcore_barrier()
  pltpu.sync_copy(acc.at[pl.ds(s * ROWS_PER_SUB, ROWS_PER_SUB)],
                  out_hbm.at[c, pl.ds(s * ROWS_PER_SUB, ROWS_PER_SUB)])


def _sc_scatter(msgs, idx_3d, zeros_tile):
  k = functools.partial(
      pl.kernel,
      mesh=_mesh,
      out_type=jax.ShapeDtypeStruct((NC, SP, D), jnp.float32),
      scratch_types=[
          pltpu.VMEM((CHUNKS, CW), jnp.int32),
          pltpu.VMEM((CW, D), jnp.float32),
          pltpu.VMEM_SHARED((SP, D), jnp.float32),
      ],
  )(_scatter_body)
  return k(msgs, idx_3d, zeros_tile)


def _gelu(x):
  return 0.5 * x * (1.0 + lax.erf(x * 0.7071067811865476))


def _scale_body(g_ref, bw_ref, msgs_ref, s0_ref, acc):
  i = pl.program_id(0)

  @pl.when(i == 0)
  def _():
    acc[...] = jnp.zeros_like(acc)

  g = g_ref[...]
  nrm = jnp.sqrt(jnp.sum(g * g, axis=1, keepdims=True))
  msgs_ref[...] = g * (bw_ref[...] / jnp.maximum(nrm, 1e-12))
  ridx = i * TILE + lax.broadcasted_iota(jnp.int32, (TILE, 1), 0)
  gm = jnp.where(ridx < BE, g, 0.0)
  acc[0:1, :] += jnp.sum(gm, axis=0, keepdims=True)
  acc[1:2, :] += jnp.sum(gm * gm, axis=0, keepdims=True)

  @pl.when(i == NT - 1)
  def _():
    s0_ref[...] = acc[...]


def _tc_scale(g, bwp):
  return pl.pallas_call(
      _scale_body,
      grid=(NT,),
      in_specs=[
          pl.BlockSpec((TILE, D), lambda i: (i, 0)),
          pl.BlockSpec((TILE, 1), lambda i: (i, 0)),
      ],
      out_specs=[
          pl.BlockSpec((TILE, D), lambda i: (i, 0)),
          pl.BlockSpec((8, D), lambda i: (0, 0)),
      ],
      out_shape=[
          jax.ShapeDtypeStruct((BEP, D), jnp.float32),
          jax.ShapeDtypeStruct((8, D), jnp.float32),
      ],
      scratch_shapes=[pltpu.VMEM((8, D), jnp.float32)],
  )(g, bwp)


def _stats_body(gs_ref, s1_ref, acc):
  i = pl.program_id(0)

  @pl.when(i == 0)
  def _():
    acc[...] = jnp.zeros_like(acc)

  g = gs_ref[...]
  ridx = i * TILE + lax.broadcasted_iota(jnp.int32, (TILE, 1), 0)
  gm = jnp.where(ridx < BE, g, 0.0)
  acc[0:1, :] += jnp.sum(gm, axis=0, keepdims=True)
  acc[1:2, :] += jnp.sum(gm * gm, axis=0, keepdims=True)

  @pl.when(i == NT - 1)
  def _():
    s1_ref[...] = acc[...]


def _tc_stats(gs):
  return pl.pallas_call(
      _stats_body,
      grid=(NT,),
      in_specs=[pl.BlockSpec((TILE, D), lambda i: (i, 0))],
      out_specs=pl.BlockSpec((8, D), lambda i: (0, 0)),
      out_shape=jax.ShapeDtypeStruct((8, D), jnp.float32),
      scratch_shapes=[pltpu.VMEM((8, D), jnp.float32)],
  )(gs)


def _snode_mlp_body(p_ref, w1_ref, b1_ref, w2_ref, b2_ref, out_ref):
  x = p_ref[0] + p_ref[1]
  h = _gelu(jnp.dot(x, w1_ref[...], preferred_element_type=jnp.float32)
            + b1_ref[...])
  out_ref[...] = _gelu(
      jnp.dot(h, w2_ref[...], preferred_element_type=jnp.float32)
      + b2_ref[...])


def _tc_snode_mlp(partials, w1, b1, w2, b2):
  return pl.pallas_call(
      _snode_mlp_body,
      out_shape=jax.ShapeDtypeStruct((SP, D), jnp.float32),
  )(partials, w1, b1.reshape(1, -1), w2, b2.reshape(1, -1))


def _cls_body(g_ref, gs_ref, s0_ref, s1_ref, bng_ref, bnb_ref,
              w1_ref, b1_ref, w2_ref, b2_ref, w3_ref, b3_ref, out_ref):
  inv = 1.0 / BE
  mu0 = s0_ref[0:1, :] * inv
  va0 = s0_ref[1:2, :] * inv - mu0 * mu0
  mu1 = s1_ref[0:1, :] * inv
  va1 = s1_ref[1:2, :] * inv - mu1 * mu1
  sc0 = bng_ref[:, 0:D] * lax.rsqrt(va0 + 1e-5)
  sc1 = bng_ref[:, D:2 * D] * lax.rsqrt(va1 + 1e-5)
  x0 = (g_ref[...] - mu0) * sc0 + bnb_ref[:, 0:D]
  x1 = (gs_ref[...] - mu1) * sc1 + bnb_ref[:, D:2 * D]
  x = jnp.concatenate([x0, x1], axis=1)
  h = _gelu(jnp.dot(x, w1_ref[...], preferred_element_type=jnp.float32)
            + b1_ref[...])
  h = _gelu(jnp.dot(h, w2_ref[...], preferred_element_type=jnp.float32)
            + b2_ref[...])
  out_ref[...] = (jnp.dot(h, w3_ref[...], preferred_element_type=jnp.float32)
                  + b3_ref[0, 0])


def _tc_classifier(g, gs, s0, s1, bng, bnb, w1, b1, w2, b2, w3, b3):
  return pl.pallas_call(
      _cls_body,
      grid=(NT,),
      in_specs=[
          pl.BlockSpec((TILE, D), lambda i: (i, 0)),
          pl.BlockSpec((TILE, D), lambda i: (i, 0)),
          pl.BlockSpec((8, D), lambda i: (0, 0)),
          pl.BlockSpec((8, D), lambda i: (0, 0)),
          pl.BlockSpec((1, 2 * D), lambda i: (0, 0)),
          pl.BlockSpec((1, 2 * D), lambda i: (0, 0)),
          pl.BlockSpec((2 * D, 2 * D), lambda i: (0, 0)),
          pl.BlockSpec((1, 2 * D), lambda i: (0, 0)),
          pl.BlockSpec((2 * D, 2 * D), lambda i: (0, 0)),
          pl.BlockSpec((1, 2 * D), lambda i: (0, 0)),
          pl.BlockSpec((2 * D, 1), lambda i: (0, 0)),
          pl.BlockSpec((1, 1), lambda i: (0, 0)),
      ],
      out_specs=pl.BlockSpec((TILE, 1), lambda i: (i, 0)),
      out_shape=jax.ShapeDtypeStruct((BEP, 1), jnp.float32),
  )(g, gs, s0, s1, bng, bnb, w1, b1, w2, b2, w3, b3)


def kernel(nodes, edges, semb, graph, bgraph, bweights, sgraph, sweights,
           snode_W1, snode_b1, snode_W2, snode_b2, sedge_W1, sedge_b1,
           sedge_W2, sedge_b2, cls_W1, cls_b1, cls_W2, cls_b2, cls_W3,
           cls_b3, bn_g, bn_b):
  pad = BEP - BE
  bg0 = jnp.concatenate([bgraph[0], jnp.zeros((pad,), jnp.int32)])
  bg1 = jnp.concatenate([bgraph[1], jnp.zeros((pad,), jnp.int32)])
  bg0_3d = bg0.reshape(NW, CHUNKS, CW)
  bg1_3d = bg1.reshape(NW, CHUNKS, CW)
  bwp = jnp.concatenate([bweights, jnp.zeros((pad, 1), jnp.float32)])
  zeros_tile = jnp.zeros((ROWS_PER_SUB, D), jnp.float32)

  g = _sc_gather(nodes, bg0_3d)
  msgs, s0 = _tc_scale(g, bwp)
  partials = _sc_scatter(msgs, bg1_3d, zeros_tile)
  snodes = _tc_snode_mlp(partials, snode_W1, snode_b1, snode_W2, snode_b2)
  gs = _sc_gather(snodes, bg1_3d)
  s1 = _tc_stats(gs)
  logits = _tc_classifier(g, gs, s0, s1, bn_g.reshape(1, -1),
                          bn_b.reshape(1, -1), cls_W1, cls_b1.reshape(1, -1),
                          cls_W2, cls_b2.reshape(1, -1), cls_W3,
                          cls_b3.reshape(1, 1))
  return logits[:BE, 0]


# speedup vs baseline: 1.5979x; 1.0663x over previous
import functools

import jax
import jax.numpy as jnp
from jax import lax
from jax.experimental import pallas as pl
from jax.experimental.pallas import tpu as pltpu
from jax.experimental.pallas import tpu_sc as plsc

NC, NS, LANES = 2, 16, 16
NW = NC * NS
BE = 40000
BEP = 40960
CHUNKS = 10
CW = 128
PER_W = CHUNKS * CW
D = 128
SP = 1024
ROWS_PER_SUB = SP // NS
TILE = 512
NT = BEP // TILE

_mesh = plsc.VectorSubcoreMesh(core_axis_name="c", subcore_axis_name="s")


def _gather_body(table_hbm, idx_hbm, out_hbm, idx_v, rows_v, gsem, wsem):
  wid = lax.axis_index("s") * NC + lax.axis_index("c")
  pltpu.sync_copy(idx_hbm.at[wid], idx_v)
  g = [None, None]
  w = [None, None]
  g[0] = pltpu.async_copy(table_hbm.at[idx_v.at[0]], rows_v.at[0], gsem)
  for j in range(CHUNKS):
    b, nb = j % 2, (j + 1) % 2
    g[b].wait()
    if j + 1 < CHUNKS:
      if w[nb] is not None:
        w[nb].wait()
      g[nb] = pltpu.async_copy(table_hbm.at[idx_v.at[j + 1]], rows_v.at[nb],
                               gsem)
    w[b] = pltpu.async_copy(
        rows_v.at[b], out_hbm.at[pl.ds(wid * PER_W + j * CW, CW)], wsem)
  for h in w:
    if h is not None:
      h.wait()


def _sc_gather(table, idx_3d):
  k = functools.partial(
      pl.kernel,
      mesh=_mesh,
      out_type=jax.ShapeDtypeStruct((BEP, D), jnp.float32),
      scratch_types=[
          pltpu.VMEM((CHUNKS, CW), jnp.int32),
          pltpu.VMEM((2, CW, D), jnp.float32),
          pltpu.SemaphoreType.DMA,
          pltpu.SemaphoreType.DMA,
      ],
  )(_gather_body)
  return k(table, idx_3d)


def _scatter_body(msgs_hbm, idx_hbm, zeros_hbm, out_hbm, idx_v, rows_v, acc):
  c = lax.axis_index("c")
  s = lax.axis_index("s")
  wid = s * NC + c
  pltpu.sync_copy(zeros_hbm, acc.at[pl.ds(s * ROWS_PER_SUB, ROWS_PER_SUB)])
  pltpu.sync_copy(idx_hbm.at[wid], idx_v)
  plsc.subcore_barrier()
  for j in range(CHUNKS):
    pltpu.sync_copy(msgs_hbm.at[pl.ds(wid * PER_W + j * CW, CW)], rows_v)
    pltpu.sync_copy(rows_v, acc.at[idx_v.at[j]], add=True)
  plsc.subcore_barrier()
  pltpu.sync_copy(acc.at[pl.ds(s * ROWS_PER_SUB, ROWS_PER_SUB)],
                  out_hbm.at[c, pl.ds(s * ROWS_PER_SUB, ROWS_PER_SUB)])


def _sc_scatter(msgs, idx_3d, zeros_tile):
  k = functools.partial(
      pl.kernel,
      mesh=_mesh,
      out_type=jax.ShapeDtypeStruct((NC, SP, D), jnp.float32),
      scratch_types=[
          pltpu.VMEM((CHUNKS, CW), jnp.int32),
          pltpu.VMEM((CW, D), jnp.float32),
          pltpu.VMEM_SHARED((SP, D), jnp.float32),
      ],
  )(_scatter_body)
  return k(msgs, idx_3d, zeros_tile)


def _gelu(x):
  return 0.5 * x * (1.0 + lax.erf(x * 0.7071067811865476))


def _scale_body(g_ref, bw_ref, bg1_ref, msgs_ref, s0_ref, c1_ref, acc, acc_c):
  i = pl.program_id(0)

  @pl.when(i == 0)
  def _():
    acc[...] = jnp.zeros_like(acc)
    acc_c[...] = jnp.zeros_like(acc_c)

  g = g_ref[...]
  nrm = jnp.sqrt(jnp.sum(g * g, axis=1, keepdims=True))
  msgs_ref[...] = g * (bw_ref[...] / jnp.maximum(nrm, 1e-12))
  ridx = i * TILE + lax.broadcasted_iota(jnp.int32, (TILE, 1), 0)
  real = ridx < BE
  gm = jnp.where(real, g, 0.0)
  acc[0:1, :] += jnp.sum(gm, axis=0, keepdims=True)
  acc[1:2, :] += jnp.sum(gm * gm, axis=0, keepdims=True)
  bins = lax.broadcasted_iota(jnp.int32, (1, SP), 1)
  onehot = jnp.where((bg1_ref[...] == bins) & real, 1.0, 0.0)
  acc_c[0:1, :] += jnp.sum(onehot, axis=0, keepdims=True)

  @pl.when(i == NT - 1)
  def _():
    s0_ref[...] = acc[...]
    c1_ref[...] = acc_c[...]


def _tc_scale(g, bwp, bg1_col):
  return pl.pallas_call(
      _scale_body,
      grid=(NT,),
      in_specs=[
          pl.BlockSpec((TILE, D), lambda i: (i, 0)),
          pl.BlockSpec((TILE, 1), lambda i: (i, 0)),
          pl.BlockSpec((TILE, 1), lambda i: (i, 0)),
      ],
      out_specs=[
          pl.BlockSpec((TILE, D), lambda i: (i, 0)),
          pl.BlockSpec((8, D), lambda i: (0, 0)),
          pl.BlockSpec((8, SP), lambda i: (0, 0)),
      ],
      out_shape=[
          jax.ShapeDtypeStruct((BEP, D), jnp.float32),
          jax.ShapeDtypeStruct((8, D), jnp.float32),
          jax.ShapeDtypeStruct((8, SP), jnp.float32),
      ],
      scratch_shapes=[pltpu.VMEM((8, D), jnp.float32),
                      pltpu.VMEM((8, SP), jnp.float32)],
  )(g, bwp, bg1_col)


def _snode_mlp_body(p_ref, c_ref, w1_ref, b1_ref, w2_ref, b2_ref,
                    out_ref, s1_ref):
  x = p_ref[0] + p_ref[1]
  h = _gelu(jnp.dot(x, w1_ref[...], preferred_element_type=jnp.float32)
            + b1_ref[...])
  sn = _gelu(jnp.dot(h, w2_ref[...], preferred_element_type=jnp.float32)
             + b2_ref[...])
  out_ref[...] = sn
  c1 = c_ref[0:1, :]
  s1s = jnp.dot(c1, sn, preferred_element_type=jnp.float32)
  s1q = jnp.dot(c1, sn * sn, preferred_element_type=jnp.float32)
  s1_ref[...] = jnp.concatenate(
      [s1s, s1q, jnp.zeros((6, D), jnp.float32)], axis=0)


def _tc_snode_mlp(partials, c1_hist, w1, b1, w2, b2):
  return pl.pallas_call(
      _snode_mlp_body,
      out_shape=[
          jax.ShapeDtypeStruct((SP, D), jnp.float32),
          jax.ShapeDtypeStruct((8, D), jnp.float32),
      ],
  )(partials, c1_hist, w1, b1.reshape(1, -1), w2, b2.reshape(1, -1))


def _cls_body(g_ref, gs_ref, s0_ref, s1_ref, bng_ref, bnb_ref,
              w1_ref, b1_ref, w2_ref, b2_ref, w3_ref, b3_ref, out_ref):
  inv = 1.0 / BE
  mu0 = s0_ref[0:1, :] * inv
  va0 = s0_ref[1:2, :] * inv - mu0 * mu0
  mu1 = s1_ref[0:1, :] * inv
  va1 = s1_ref[1:2, :] * inv - mu1 * mu1
  sc0 = bng_ref[:, 0:D] * lax.rsqrt(va0 + 1e-5)
  sc1 = bng_ref[:, D:2 * D] * lax.rsqrt(va1 + 1e-5)
  x0 = (g_ref[...] - mu0) * sc0 + bnb_ref[:, 0:D]
  x1 = (gs_ref[...] - mu1) * sc1 + bnb_ref[:, D:2 * D]
  x = jnp.concatenate([x0, x1], axis=1)
  h = _gelu(jnp.dot(x, w1_ref[...], preferred_element_type=jnp.float32)
            + b1_ref[...])
  h = _gelu(jnp.dot(h, w2_ref[...], preferred_element_type=jnp.float32)
            + b2_ref[...])
  out_ref[...] = (jnp.dot(h, w3_ref[...], preferred_element_type=jnp.float32)
                  + b3_ref[0, 0])


def _tc_classifier(g, gs, s0, s1, bng, bnb, w1, b1, w2, b2, w3, b3):
  return pl.pallas_call(
      _cls_body,
      grid=(NT,),
      in_specs=[
          pl.BlockSpec((TILE, D), lambda i: (i, 0)),
          pl.BlockSpec((TILE, D), lambda i: (i, 0)),
          pl.BlockSpec((8, D), lambda i: (0, 0)),
          pl.BlockSpec((8, D), lambda i: (0, 0)),
          pl.BlockSpec((1, 2 * D), lambda i: (0, 0)),
          pl.BlockSpec((1, 2 * D), lambda i: (0, 0)),
          pl.BlockSpec((2 * D, 2 * D), lambda i: (0, 0)),
          pl.BlockSpec((1, 2 * D), lambda i: (0, 0)),
          pl.BlockSpec((2 * D, 2 * D), lambda i: (0, 0)),
          pl.BlockSpec((1, 2 * D), lambda i: (0, 0)),
          pl.BlockSpec((2 * D, 1), lambda i: (0, 0)),
          pl.BlockSpec((1, 1), lambda i: (0, 0)),
      ],
      out_specs=pl.BlockSpec((TILE, 1), lambda i: (i, 0)),
      out_shape=jax.ShapeDtypeStruct((BEP, 1), jnp.float32),
  )(g, gs, s0, s1, bng, bnb, w1, b1, w2, b2, w3, b3)


def kernel(nodes, edges, semb, graph, bgraph, bweights, sgraph, sweights,
           snode_W1, snode_b1, snode_W2, snode_b2, sedge_W1, sedge_b1,
           sedge_W2, sedge_b2, cls_W1, cls_b1, cls_W2, cls_b2, cls_W3,
           cls_b3, bn_g, bn_b):
  pad = BEP - BE
  bg0 = jnp.concatenate([bgraph[0], jnp.zeros((pad,), jnp.int32)])
  bg1 = jnp.concatenate([bgraph[1], jnp.zeros((pad,), jnp.int32)])
  bg0_3d = bg0.reshape(NW, CHUNKS, CW)
  bg1_3d = bg1.reshape(NW, CHUNKS, CW)
  bwp = jnp.concatenate([bweights, jnp.zeros((pad, 1), jnp.float32)])
  zeros_tile = jnp.zeros((ROWS_PER_SUB, D), jnp.float32)

  g = _sc_gather(nodes, bg0_3d)
  msgs, s0, c1 = _tc_scale(g, bwp, bg1.reshape(BEP, 1))
  partials = _sc_scatter(msgs, bg1_3d, zeros_tile)
  snodes, s1 = _tc_snode_mlp(partials, c1, snode_W1, snode_b1,
                             snode_W2, snode_b2)
  gs = _sc_gather(snodes, bg1_3d)
  logits = _tc_classifier(g, gs, s0, s1, bn_g.reshape(1, -1),
                          bn_b.reshape(1, -1), cls_W1, cls_b1.reshape(1, -1),
                          cls_W2, cls_b2.reshape(1, -1), cls_W3,
                          cls_b3.reshape(1, 1))
  return logits[:BE, 0]
